# baseline (speedup 1.0000x reference)
import jax
import jax.numpy as jnp
from jax.experimental import pallas as pl
from jax.experimental.pallas import tpu as pltpu

_BETA = 0.5
_EPS = 1e-06
_LN2 = 0.6931471805599453

_B, _N, _C = 16, 20000, 81
_W = 2000
_NS = 8
_GB = _B // _NS
_GN = _N // _W
_LW = 2000
_LGN = _N // _LW


def _conf_kernel(*refs):
    b = pl.program_id(0)
    j = pl.program_id(1)
    ct_ref = refs[_NS]
    acc_ref = refs[_NS + 1]

    ctb = ct_ref[...]
    lanes = jax.lax.broadcasted_iota(jnp.int32, (_W, _C), 1)
    s = jnp.zeros((1, 1), jnp.float32)
    for r in range(_NS):
        p = refs[r][0]
        ct_col = ctb[:, r:r + 1]
        is_t = jnp.logical_and(lanes == ct_col, ct_col > 0)
        q = jnp.where(is_t, 1.0 - p, p)
        lg = jnp.log2(jnp.where(is_t, p, 1.0 - p) + _EPS)
        s += jnp.sum(q * q * lg).reshape(1, 1)

    @pl.when(jnp.logical_and(b == 0, j == 0))
    def _():
        acc_ref[...] = s

    @pl.when(jnp.logical_or(b != 0, j != 0))
    def _():
        acc_ref[...] += s


def _loc_kernel(lp_ref, lt_ref, ct_ref, craw_ref,
                tot_ref, conf_ref, loc_ref):
    i = pl.program_id(0)
    jn = pl.program_id(1)
    r = jax.lax.rem(i, _NS)

    ctb = ct_ref[...]
    lane_io = jax.lax.broadcasted_iota(jnp.int32, (_LW, 128), 1)
    posm = jnp.logical_and(lane_io == r, ctb > 0)
    pos_col = jnp.sum(jnp.where(posm, 1.0, 0.0), axis=1, keepdims=True)

    z = jnp.abs(lp_ref[0] - lt_ref[0])
    sl1 = jnp.where(z < 1.0, 0.5 * z * z, z - 0.5)
    srow = jnp.sum(sl1, axis=1, keepdims=True)

    loc_s = jnp.sum(srow * pos_col).reshape(1, 1)
    cnt_s = jnp.sum(pos_col).reshape(1, 1)

    first = jnp.logical_and(i == 0, jn == 0)

    @pl.when(first)
    def _():
        loc_ref[...] = loc_s
        tot_ref[...] = cnt_s

    @pl.when(jnp.logical_not(first))
    def _():
        loc_ref[...] += loc_s
        tot_ref[...] += cnt_s

    @pl.when(jnp.logical_and(i == _B - 1, jn == _LGN - 1))
    def _():
        cnt = tot_ref[0, 0]
        conf = (-0.5 * _LN2) * craw_ref[0, 0] / cnt
        loc = loc_ref[0, 0] / cnt
        conf_ref[...] = jnp.full((1, 1), conf, jnp.float32)
        loc_ref[...] = jnp.full((1, 1), loc, jnp.float32)
        tot_ref[...] = jnp.full((1, 1), _BETA * conf + (1.0 - _BETA) * loc,
                                jnp.float32)


@jax.jit
def _run(loc_preds, loc_targets, conf_preds, conf_targets):
    B, N, C = conf_preds.shape
    ct = conf_targets.astype(jnp.int32)
    ct_t = ct.reshape(_GB, _NS, N).transpose(0, 2, 1).reshape(_GB * N, _NS)
    ct_t = jnp.pad(ct_t, ((0, 0), (0, 128 - _NS)))

    conf_raw = pl.pallas_call(
        _conf_kernel,
        grid=(_GB, _GN),
        in_specs=(
            [pl.BlockSpec((1, _W, C), lambda b, j, s=s: (b * _NS + s, j, 0))
             for s in range(_NS)]
            + [pl.BlockSpec((_W, 128), lambda b, j: (b * _GN + j, 0))]
        ),
        out_specs=pl.BlockSpec((1, 1), lambda b, j: (0, 0)),
        out_shape=jax.ShapeDtypeStruct((1, 1), jnp.float32),
    )(*([conf_preds] * _NS + [ct_t]))

    out_spec = pl.BlockSpec((1, 1), lambda i, jn: (0, 0))
    tot, conf, loc = pl.pallas_call(
        _loc_kernel,
        grid=(_B, _LGN),
        in_specs=[
            pl.BlockSpec((1, _LW, 4), lambda i, jn: (i, jn, 0)),
            pl.BlockSpec((1, _LW, 4), lambda i, jn: (i, jn, 0)),
            pl.BlockSpec((_LW, 128),
                         lambda i, jn: ((i // _NS) * (_N // _LW) + jn, 0)),
            out_spec,
        ],
        out_specs=[out_spec, out_spec, out_spec],
        out_shape=[
            jax.ShapeDtypeStruct((1, 1), jnp.float32),
            jax.ShapeDtypeStruct((1, 1), jnp.float32),
            jax.ShapeDtypeStruct((1, 1), jnp.float32),
        ],
    )(loc_preds, loc_targets, ct_t, conf_raw)

    return (tot[0, 0], conf[0, 0], loc[0, 0])


def kernel(loc_preds, loc_targets, conf_preds, conf_targets):
    return _run(loc_preds, loc_targets, conf_preds, conf_targets)


# speedup vs baseline: 1.1233x; 1.1233x over previous
import jax
import jax.numpy as jnp
from jax.experimental import pallas as pl
from jax.experimental.pallas import tpu as pltpu

_BETA = 0.5
_EPS = 1e-06
_LN2 = 0.6931471805599453

_B, _N, _C = 16, 20000, 81
_W = 2000
_NS = 8
_GB = _B // _NS
_GN = _N // _W
_LW = 10000
_LGN = _N // _LW


def _conf_kernel(*refs):
    b = pl.program_id(0)
    j = pl.program_id(1)
    ct_ref = refs[_NS]
    acc_ref = refs[_NS + 1]

    ctb = ct_ref[...]
    lanes = jax.lax.broadcasted_iota(jnp.int32, (_W, _C), 1)
    s = jnp.zeros((1, 1), jnp.float32)
    for r in range(_NS):
        p = refs[r][0]
        ct_col = ctb[:, r:r + 1]
        is_t = jnp.logical_and(lanes == ct_col, ct_col > 0)
        q = jnp.where(is_t, 1.0 - p, p)
        lg = jnp.log2(jnp.where(is_t, p, 1.0 - p) + _EPS)
        s += jnp.sum(q * q * lg).reshape(1, 1)

    @pl.when(jnp.logical_and(b == 0, j == 0))
    def _():
        acc_ref[...] = s

    @pl.when(jnp.logical_or(b != 0, j != 0))
    def _():
        acc_ref[...] += s


def _loc_kernel(lp_ref, lt_ref, ct_ref, craw_ref,
                tot_ref, conf_ref, loc_ref):
    i = pl.program_id(0)
    jn = pl.program_id(1)
    r = jax.lax.rem(i, _NS)

    ctb = ct_ref[...]
    lane_io = jax.lax.broadcasted_iota(jnp.int32, (_LW, 128), 1)
    posm = jnp.logical_and(lane_io == r, ctb > 0)
    pos_col = jnp.sum(jnp.where(posm, 1.0, 0.0), axis=1, keepdims=True)

    z = jnp.abs(lp_ref[0] - lt_ref[0])
    sl1 = jnp.where(z < 1.0, 0.5 * z * z, z - 0.5)
    srow = jnp.sum(sl1, axis=1, keepdims=True)

    loc_s = jnp.sum(srow * pos_col).reshape(1, 1)
    cnt_s = jnp.sum(pos_col).reshape(1, 1)

    first = jnp.logical_and(i == 0, jn == 0)

    @pl.when(first)
    def _():
        loc_ref[...] = loc_s
        tot_ref[...] = cnt_s

    @pl.when(jnp.logical_not(first))
    def _():
        loc_ref[...] += loc_s
        tot_ref[...] += cnt_s

    @pl.when(jnp.logical_and(i == _B - 1, jn == _LGN - 1))
    def _():
        cnt = tot_ref[0, 0]
        conf = (-0.5 * _LN2) * craw_ref[0, 0] / cnt
        loc = loc_ref[0, 0] / cnt
        conf_ref[...] = jnp.full((1, 1), conf, jnp.float32)
        loc_ref[...] = jnp.full((1, 1), loc, jnp.float32)
        tot_ref[...] = jnp.full((1, 1), _BETA * conf + (1.0 - _BETA) * loc,
                                jnp.float32)


@jax.jit
def _run(loc_preds, loc_targets, conf_preds, conf_targets):
    B, N, C = conf_preds.shape
    ct = conf_targets.astype(jnp.int32)
    ct_t = ct.reshape(_GB, _NS, N).transpose(0, 2, 1).reshape(_GB * N, _NS)
    ct_t = jnp.pad(ct_t, ((0, 0), (0, 128 - _NS)))

    conf_raw = pl.pallas_call(
        _conf_kernel,
        grid=(_GB, _GN),
        in_specs=(
            [pl.BlockSpec((1, _W, C), lambda b, j, s=s: (b * _NS + s, j, 0))
             for s in range(_NS)]
            + [pl.BlockSpec((_W, 128), lambda b, j: (b * _GN + j, 0))]
        ),
        out_specs=pl.BlockSpec((1, 1), lambda b, j: (0, 0)),
        out_shape=jax.ShapeDtypeStruct((1, 1), jnp.float32),
    )(*([conf_preds] * _NS + [ct_t]))

    out_spec = pl.BlockSpec((1, 1), lambda i, jn: (0, 0))
    tot, conf, loc = pl.pallas_call(
        _loc_kernel,
        grid=(_B, _LGN),
        in_specs=[
            pl.BlockSpec((1, _LW, 4), lambda i, jn: (i, jn, 0)),
            pl.BlockSpec((1, _LW, 4), lambda i, jn: (i, jn, 0)),
            pl.BlockSpec((_LW, 128),
                         lambda i, jn: ((i // _NS) * _LGN + jn, 0)),
            out_spec,
        ],
        out_specs=[out_spec, out_spec, out_spec],
        out_shape=[
            jax.ShapeDtypeStruct((1, 1), jnp.float32),
            jax.ShapeDtypeStruct((1, 1), jnp.float32),
            jax.ShapeDtypeStruct((1, 1), jnp.float32),
        ],
    )(loc_preds, loc_targets, ct_t, conf_raw)

    return (tot[0, 0], conf[0, 0], loc[0, 0])


def kernel(loc_preds, loc_targets, conf_preds, conf_targets):
    return _run(loc_preds, loc_targets, conf_preds, conf_targets)
